# Initial kernel scaffold; baseline (speedup 1.0000x reference)
import jax
import jax.numpy as jnp
from jax.experimental import pallas as pl


def kernel(x, edge_index, edge_attr, globals_, enc_params, rec_params, dec_params):
    raise NotImplementedError("write your pallas kernel here")


# speedup vs baseline: 3.0309x; 3.0309x over previous
import jax
import jax.numpy as jnp
from jax import lax
from jax.experimental import pallas as pl
from jax.experimental.pallas import tpu as pltpu
from jax.experimental.pallas import tpu_sc as plsc

NN = 10000
NE = 320000
H = 32
NT = 32
NNP = 10240
NEP = 327680
EPT = NEP // NT
IB = 128
NIB = 16
CHUNK = IB * NIB
NCH = EPT // CHUNK
RPT = EPT // IB
NPT = NNP // 16
BLK = 4096
BLKN = 2048
GRIDN = NNP // BLKN

f32 = jnp.float32


def _dot(a, b):
    return jnp.dot(a, b, preferred_element_type=f32)


def _sp(x):
    return jnp.maximum(x, 0.0) + jnp.log1p(jnp.exp(-jnp.abs(x)))


def _node0a_body(xp, w0n, b0n, w1n, b1n, wsc, wrc, ne_o, ps_o, pr_o):
    ne = _sp(_dot(_sp(_dot(xp[...], w0n[...]) + b0n[...]), w1n[...]) + b1n[...])
    ne_o[...] = ne
    ps_o[...] = _dot(ne, wsc[...])
    pr_o[...] = _dot(ne, wrc[...])


def _node0b_body(g2, w0g, b0g, w1g, b1g, wgc, ceb, ge_o, ce_o):
    ge = _sp(_dot(_sp(_dot(g2[...], w0g[...]) + b0g[...]), w1g[...]) + b1g[...])
    ge_o[...] = ge
    ce_o[...] = _dot(ge, wgc[...]) + ceb[...]


def _edge0_body(ea, gs, gr, w0, b0, w1, b1, ab, w1e, b1e, ce, ee_o, e1_o):
    ee = _sp(_dot(_sp(_dot(ea[...], w0[...]) + b0[...]), w1[...]) + b1[...])
    ee_o[...] = ee
    h = _sp(_dot(ee, ab[...]) + gs[...] + gr[...] + ce[...])
    e1_o[...] = _sp(_dot(h, w1e[...]) + b1e[...])


def _edge1_body(e, ee, gs, gr, wa, wb, w1e, b1e, ce, out_o):
    h = _sp(_dot(e[...], wa[...]) + _dot(ee[...], wb[...])
            + gs[...] + gr[...] + ce[...])
    out_o[...] = _sp(_dot(h, w1e[...]) + b1e[...])


def _node_block(n, ne, agg1, agg2, g, ge, wn1, wn2, wa, wgn1, wgn2, b0n,
                w1n, b1n):
    agg = agg1[...] + agg2[...]
    h = _sp(_dot(n[...], wn1[...]) + _dot(ne[...], wn2[...])
            + _dot(agg, wa[...]) + _dot(g[...], wgn1[...])
            + _dot(ge[...], wgn2[...]) + b0n[...])
    n2 = _sp(_dot(h, w1n[...]) + b1n[...])
    rid = lax.broadcasted_iota(jnp.int32, (BLKN, H), 0) \
        + pl.program_id(0) * BLKN
    m = rid < NN
    sn_b = jnp.sum(jnp.where(m, n2, 0.0), axis=0, keepdims=True)
    se_b = jnp.sum(jnp.where(m, agg, 0.0), axis=0, keepdims=True)
    return n2, sn_b, se_b


def _nodea_body(n, ne, agg1, agg2, g, ge,
                wn1, wn2, wa, wgn1, wgn2, b0n, w1n, b1n,
                wsa, wsb, wra, wrb,
                n_o, ps_o, pr_o, sn_o, se_o):
    n2, sn_b, se_b = _node_block(n, ne, agg1, agg2, g, ge, wn1, wn2, wa,
                                 wgn1, wgn2, b0n, w1n, b1n)
    n_o[...] = n2
    ps_o[...] = _dot(n2, wsa[...]) + _dot(ne[...], wsb[...])
    pr_o[...] = _dot(n2, wra[...]) + _dot(ne[...], wrb[...])
    i = pl.program_id(0)

    @pl.when(i == 0)
    def _():
        sn_o[...] = sn_b
        se_o[...] = se_b

    @pl.when(i > 0)
    def _():
        sn_o[...] += sn_b
        se_o[...] += se_b


def _nodefa_body(n, ne, agg1, agg2, g, ge,
                 wn1, wn2, wa, wgn1, wgn2, b0n, w1n, b1n,
                 sn_o, se_o):
    _, sn_b, se_b = _node_block(n, ne, agg1, agg2, g, ge, wn1, wn2, wa,
                                wgn1, wgn2, b0n, w1n, b1n)
    i = pl.program_id(0)

    @pl.when(i == 0)
    def _():
        sn_o[...] = sn_b
        se_o[...] = se_b

    @pl.when(i > 0)
    def _():
        sn_o[...] += sn_b
        se_o[...] += se_b


def _nodeb_body(sn, se, g, ge, wgsn, wgse, wgg1, wgg2, b0g, w1g, b1g,
                wga, wgb, b0e, g_o, ce_o):
    hg = _sp(_dot(sn[...], wgsn[...]) + _dot(se[...], wgse[...])
             + _dot(g[...], wgg1[...]) + _dot(ge[...], wgg2[...]) + b0g[...])
    g2 = _sp(_dot(hg, w1g[...]) + b1g[...])
    g_o[...] = g2
    ce_o[...] = _dot(g2, wga[...]) + _dot(ge[...], wgb[...]) + b0e[...]


def _nodefb_body(sn, se, g, ge, wgsn, wgse, wgg1, wgg2, b0g, w1g, b1g,
                 d0, db0, d1, db1, d2, db2, out_o):
    hg = _sp(_dot(sn[...], wgsn[...]) + _dot(se[...], wgse[...])
             + _dot(g[...], wgg1[...]) + _dot(ge[...], wgg2[...]) + b0g[...])
    g2 = _sp(_dot(hg, w1g[...]) + b1g[...])
    d = _sp(_dot(g2, d0[...]) + db0[...])
    d = _sp(_dot(d, d1[...]) + db1[...])
    out_o[...] = _dot(d, d2[...]) + db2[...]


def _gather_body(ps, pr, si, ri, gs_o, gr_o, idx_v, rows_v, sem):
    c = lax.axis_index("c")
    s = lax.axis_index("s")
    wid = s * 2 + c

    def one(tab, ih, oh):
        def chunk(ci, carry):
            r0 = wid * RPT + ci * NIB
            e0 = wid * EPT + ci * CHUNK
            pltpu.sync_copy(ih.at[pl.ds(r0, NIB)], idx_v)
            descs = [
                pltpu.async_copy(tab.at[idx_v.at[j]],
                                 rows_v.at[pl.ds(j * IB, IB)], sem)
                for j in range(NIB)
            ]
            for d in descs:
                d.wait()
            pltpu.sync_copy(rows_v, oh.at[pl.ds(e0, CHUNK)])
            return carry

        lax.fori_loop(0, NCH, chunk, 0)

    one(ps, si, gs_o)
    one(pr, ri, gr_o)


def _scatter_body(e, ri, z, out_o, idx_v, rows_v, acc, sem):
    c = lax.axis_index("c")
    s = lax.axis_index("s")
    wid = s * 2 + c
    pltpu.sync_copy(z.at[pl.ds(s * NPT, NPT)], acc.at[pl.ds(s * NPT, NPT)])
    plsc.subcore_barrier()

    def chunk(ci, carry):
        r0 = wid * RPT + ci * NIB
        e0 = wid * EPT + ci * CHUNK
        pltpu.sync_copy(ri.at[pl.ds(r0, NIB)], idx_v)
        pltpu.sync_copy(e.at[pl.ds(e0, CHUNK)], rows_v)
        descs = [
            pltpu.async_copy(rows_v.at[pl.ds(j * IB, IB)],
                             acc.at[idx_v.at[j]], sem, add=True)
            for j in range(NIB)
        ]
        for d in descs:
            d.wait()
        return carry

    lax.fori_loop(0, NCH, chunk, 0)
    plsc.subcore_barrier()
    pltpu.sync_copy(acc.at[pl.ds(s * NPT, NPT)],
                    out_o.at[pl.ds(c * NNP + s * NPT, NPT)])


_SC_CACHE = {}


def _sc_kernels():
    if "g" not in _SC_CACHE:
        mesh = plsc.VectorSubcoreMesh(core_axis_name="c",
                                      subcore_axis_name="s")
        scp = pltpu.CompilerParams(use_tc_tiling_on_sc=False)
        _SC_CACHE["g"] = pl.kernel(
            _gather_body,
            out_type=[jax.ShapeDtypeStruct((NEP, H), f32),
                      jax.ShapeDtypeStruct((NEP, H), f32)],
            mesh=mesh,
            compiler_params=scp,
            scratch_types=[pltpu.VMEM((NIB, IB), jnp.int32),
                           pltpu.VMEM((CHUNK, H), f32),
                           pltpu.SemaphoreType.DMA],
        )
        _SC_CACHE["s"] = pl.kernel(
            _scatter_body,
            out_type=jax.ShapeDtypeStruct((2 * NNP, H), f32),
            mesh=mesh,
            compiler_params=scp,
            scratch_types=[pltpu.VMEM((NIB, IB), jnp.int32),
                           pltpu.VMEM((CHUNK, H), f32),
                           pltpu.VMEM_SHARED((NNP, H), f32),
                           pltpu.SemaphoreType.DMA],
        )
    return _SC_CACHE["g"], _SC_CACHE["s"]


def _full(a):
    return pl.BlockSpec(a.shape, lambda i: (0, 0))


def _eblk(d):
    return pl.BlockSpec((BLK, d), lambda i: (i, 0))


def _nblk(d):
    return pl.BlockSpec((BLKN, d), lambda i: (i, 0))


_row = pl.BlockSpec((1, H), lambda i: (0, 0))
_arb = pltpu.CompilerParams(dimension_semantics=("arbitrary",))


def kernel(x, edge_index, edge_attr, globals_, enc_params, rec_params,
           dec_params):
    r2 = lambda v: v.reshape(1, -1)
    dummy = (NN + (jnp.arange(NEP - NE, dtype=jnp.int32) % (NNP - NN)))
    si2 = jnp.concatenate([edge_index[0].astype(jnp.int32), dummy]
                          ).reshape(NEP // IB, IB)
    ri2 = jnp.concatenate([edge_index[1].astype(jnp.int32), dummy]
                          ).reshape(NEP // IB, IB)
    ea_p = jnp.pad(edge_attr, ((0, NEP - NE), (0, 0)))
    x_p = jnp.pad(x, ((0, NNP - NN), (0, 0)))
    g2 = globals_.reshape(1, -1).astype(f32)
    zeros = jnp.zeros((NNP, H), f32)

    (w0e, b0e), (w1e, b1e) = enc_params[0]
    (w0n, b0n), (w1n, b1n) = enc_params[1]
    (w0g, b0g), (w1g, b1g) = enc_params[2]

    def pe_slices(pe):
        (w0, b0), (w1, b1) = pe
        return dict(A=w0[0:32], B=w0[32:64], WsA=w0[64:96], WsB=w0[96:128],
                    WrA=w0[128:160], WrB=w0[160:192], WgA=w0[192:224],
                    WgB=w0[224:256], b0=r2(b0), w1=w1, b1=r2(b1))

    def pn_slices(pn):
        (w0, b0), (w1, b1) = pn
        return dict(wn1=w0[0:32], wn2=w0[32:64], wa=w0[64:96],
                    wgn1=w0[96:128], wgn2=w0[128:160], b0=r2(b0), w1=w1,
                    b1=r2(b1))

    def pg_slices(pg):
        (w0, b0), (w1, b1) = pg
        return dict(wgsn=w0[0:32], wgse=w0[32:64], wgg1=w0[64:96],
                    wgg2=w0[96:128], b0=r2(b0), w1=w1, b1=r2(b1))

    pe0, pn0, pg0 = rec_params[0]
    pe1, pn1, pg1 = rec_params[1]
    e0s, e1s = pe_slices(pe0), pe_slices(pe1)
    n0s, n1s = pn_slices(pn0), pn_slices(pn1)
    g0s, g1s = pg_slices(pg0), pg_slices(pg1)

    _gather2, _scatter2 = _sc_kernels()

    nshape = jax.ShapeDtypeStruct((NNP, H), f32)
    rshape = jax.ShapeDtypeStruct((1, H), f32)

    wargs = (w0n, r2(b0n), w1n, r2(b1n),
             e0s["WsA"] + e0s["WsB"], e0s["WrA"] + e0s["WrB"])
    n_enc, ps0, pr0 = pl.pallas_call(
        _node0a_body,
        grid=(GRIDN,),
        in_specs=[_nblk(128)] + [_full(a) for a in wargs],
        out_specs=[_nblk(H)] * 3,
        out_shape=[nshape] * 3,
        compiler_params=_arb,
    )(x_p, *wargs)

    g_enc, ce0 = pl.pallas_call(
        _node0b_body,
        out_shape=[rshape, rshape],
    )(g2, w0g, r2(b0g), w1g, r2(b1g), e0s["WgA"] + e0s["WgB"], e0s["b0"])

    gs0, gr0 = _gather2(ps0, pr0, si2, ri2)

    wargs0 = (w0e, r2(b0e), w1e, r2(b1e), e0s["A"] + e0s["B"], e0s["w1"],
              e0s["b1"])
    e_enc, e1 = pl.pallas_call(
        _edge0_body,
        grid=(NEP // BLK,),
        in_specs=[_eblk(4), _eblk(H), _eblk(H)] + [_full(a) for a in wargs0]
                 + [_full(ce0)],
        out_specs=[_eblk(H), _eblk(H)],
        out_shape=[jax.ShapeDtypeStruct((NEP, H), f32),
                   jax.ShapeDtypeStruct((NEP, H), f32)],
        compiler_params=_arb,
    )(ea_p, gs0, gr0, *wargs0, ce0)

    aggp = _scatter2(e1, ri2, zeros)

    wargsn = (n0s["wn1"], n0s["wn2"], n0s["wa"], n0s["wgn1"], n0s["wgn2"],
              n0s["b0"], n0s["w1"], n0s["b1"],
              e1s["WsA"], e1s["WsB"], e1s["WrA"], e1s["WrB"])
    n1, ps1, pr1, sn0, se0 = pl.pallas_call(
        _nodea_body,
        grid=(GRIDN,),
        in_specs=[_nblk(H), _nblk(H),
                  pl.BlockSpec((BLKN, H), lambda i: (i, 0)),
                  pl.BlockSpec((BLKN, H), lambda i: (i + GRIDN, 0)),
                  _row, _row] + [_full(a) for a in wargsn],
        out_specs=[_nblk(H), _nblk(H), _nblk(H), _row, _row],
        out_shape=[nshape, nshape, nshape, rshape, rshape],
        compiler_params=_arb,
    )(n_enc, n_enc, aggp, aggp, g_enc, g_enc, *wargsn)

    g1, ce1 = pl.pallas_call(
        _nodeb_body,
        out_shape=[rshape, rshape],
    )(sn0, se0, g_enc, g_enc,
      g0s["wgsn"], g0s["wgse"], g0s["wgg1"], g0s["wgg2"], g0s["b0"],
      g0s["w1"], g0s["b1"],
      e1s["WgA"], e1s["WgB"], e1s["b0"])

    gs1, gr1 = _gather2(ps1, pr1, si2, ri2)

    wargs1 = (e1s["A"], e1s["B"], e1s["w1"], e1s["b1"])
    e2 = pl.pallas_call(
        _edge1_body,
        grid=(NEP // BLK,),
        in_specs=[_eblk(H)] * 4 + [_full(a) for a in wargs1] + [_full(ce1)],
        out_specs=_eblk(H),
        out_shape=jax.ShapeDtypeStruct((NEP, H), f32),
        compiler_params=_arb,
    )(e1, e_enc, gs1, gr1, *wargs1, ce1)

    aggp2 = _scatter2(e2, ri2, zeros)

    wargsf = (n1s["wn1"], n1s["wn2"], n1s["wa"], n1s["wgn1"], n1s["wgn2"],
              n1s["b0"], n1s["w1"], n1s["b1"])
    sn1, se1 = pl.pallas_call(
        _nodefa_body,
        grid=(GRIDN,),
        in_specs=[_nblk(H), _nblk(H),
                  pl.BlockSpec((BLKN, H), lambda i: (i, 0)),
                  pl.BlockSpec((BLKN, H), lambda i: (i + GRIDN, 0)),
                  _row, _row] + [_full(a) for a in wargsf],
        out_specs=[_row, _row],
        out_shape=[rshape, rshape],
        compiler_params=_arb,
    )(n1, n_enc, aggp2, aggp2, g1, g_enc, *wargsf)

    (d0, db0), (d1, db1), (d2, db2) = dec_params
    out = pl.pallas_call(
        _nodefb_body,
        out_shape=jax.ShapeDtypeStruct((1, 1), f32),
    )(sn1, se1, g1, g_enc,
      g1s["wgsn"], g1s["wgse"], g1s["wgg1"], g1s["wgg2"], g1s["b0"],
      g1s["w1"], g1s["b1"],
      d0, r2(db0), d1, r2(db1), d2, r2(db2))

    return out.reshape(())
